# Initial kernel scaffold; baseline (speedup 1.0000x reference)
import jax
import jax.numpy as jnp
from jax.experimental import pallas as pl


def kernel(x, edge_index, batch, edge_attr, pos, W_msg1, b_msg1, W_upd1, b_upd1, W_msg2, b_msg2, W_upd2, b_upd2, W_out, b_out):
    raise NotImplementedError("write your pallas kernel here")


# speedup vs baseline: 1.4034x; 1.4034x over previous
import functools

import jax
import jax.numpy as jnp
from jax import lax
from jax.experimental import pallas as pl
from jax.experimental.pallas import tpu as pltpu
from jax.experimental.pallas import tpu_sc as plsc

G = 64
NC = 2
NS = 16
CH = 96
LN = 128


def _mm_stacked_body(x_ref, w_ref, b_ref, o_ref):
    o_ref[...] = (
        jnp.dot(x_ref[...], w_ref[0], preferred_element_type=jnp.float32)
        + b_ref[0]
    )


def _stacked_matmul(xp, wcat, bcat, bm, ln=LN):
    m, k = xp.shape
    nj = wcat.shape[1] // ln
    ni = m // bm
    w3 = wcat.reshape(k, nj, ln).transpose(1, 0, 2)
    return pl.pallas_call(
        _mm_stacked_body,
        grid=(nj, ni),
        in_specs=[
            pl.BlockSpec((bm, k), lambda j, i: (i, 0)),
            pl.BlockSpec((1, k, ln), lambda j, i: (j, 0, 0)),
            pl.BlockSpec((1, 1, ln), lambda j, i: (j, 0, 0)),
        ],
        out_specs=pl.BlockSpec((bm, ln), lambda j, i, ni=ni: (j * ni + i, 0)),
        out_shape=jax.ShapeDtypeStruct((nj * m, ln), jnp.float32),
    )(xp, w3, bcat.reshape(nj, 1, ln))


def _upd_body(h_ref, alo_ref, ahi_ref, wh_ref, wlo_ref, whi_ref, b_ref,
              o_ref):
    acc = jnp.dot(h_ref[...], wh_ref[...], preferred_element_type=jnp.float32)
    acc += jnp.dot(alo_ref[...], wlo_ref[...],
                   preferred_element_type=jnp.float32)
    acc += jnp.dot(ahi_ref[...], whi_ref[...],
                   preferred_element_type=jnp.float32)
    o_ref[...] = jnp.maximum(acc + b_ref[0], 0.0)


def _update_matmul(h, agg, w_upd, b_upd, bm):
    npad, fh = h.shape
    hout = w_upd.shape[1]
    ni = npad // bm
    nc = hout // LN
    wh = w_upd[:fh]
    wlo = w_upd[fh:fh + LN]
    whi = w_upd[fh + LN:]
    return pl.pallas_call(
        _upd_body,
        grid=(nc, ni),
        in_specs=[
            pl.BlockSpec((bm, fh), lambda c, i: (i, 0)),
            pl.BlockSpec((bm, LN), lambda c, i: (i, 0)),
            pl.BlockSpec((bm, LN), lambda c, i, ni=ni: (ni + i, 0)),
            pl.BlockSpec((fh, LN), lambda c, i: (0, c)),
            pl.BlockSpec((LN, LN), lambda c, i: (0, c)),
            pl.BlockSpec((LN, LN), lambda c, i: (0, c)),
            pl.BlockSpec((1, 1, LN), lambda c, i: (c, 0, 0)),
        ],
        out_specs=pl.BlockSpec((bm, LN), lambda c, i: (i, c)),
        out_shape=jax.ShapeDtypeStruct((npad, hout), jnp.float32),
    )(h, agg, agg, wh, wlo, whi, b_upd.reshape(nc, 1, LN))


def _pool_body(b3_ref, h_ref, sum_ref, cnt_ref):
    i = pl.program_id(0)

    @pl.when(i == 0)
    def _():
        sum_ref[...] = jnp.zeros_like(sum_ref)
        cnt_ref[...] = jnp.zeros_like(cnt_ref)

    bids = b3_ref[0]
    ids = lax.broadcasted_iota(jnp.int32, (G, bids.shape[1]), 0)
    oh = (bids == ids).astype(jnp.float32)
    sum_ref[...] += jnp.dot(oh, h_ref[...], preferred_element_type=jnp.float32)
    cnt_ref[...] += jnp.sum(oh, axis=1, keepdims=True)


def _pool(batch, h, n, bn):
    nb = n // bn
    fh = h.shape[1]
    batch3 = batch.reshape(nb, 1, bn)
    return pl.pallas_call(
        _pool_body,
        grid=(nb,),
        in_specs=[
            pl.BlockSpec((1, 1, bn), lambda i: (i, 0, 0)),
            pl.BlockSpec((bn, fh), lambda i: (i, 0)),
        ],
        out_specs=[
            pl.BlockSpec((G, fh), lambda i: (0, 0)),
            pl.BlockSpec((G, LN), lambda i: (0, 0)),
        ],
        out_shape=[
            jax.ShapeDtypeStruct((G, fh), jnp.float32),
            jax.ShapeDtypeStruct((G, LN), jnp.float32),
        ],
    )(batch3, h)


def _head_body(s_ref, c_ref, w_ref, b_ref, o_ref):
    cnt = c_ref[:, :1]
    pooled = s_ref[...] / jnp.clip(cnt, 1.0, None)
    o_ref[...] = (
        jnp.dot(pooled, w_ref[...], preferred_element_type=jnp.float32)
        + b_ref[0:1, :]
    )


def _head(sums, cnts, w_out_p, b_out_p):
    fh = sums.shape[1]
    return pl.pallas_call(
        _head_body,
        out_shape=jax.ShapeDtypeStruct((G, LN), jnp.float32),
    )(sums, cnts, w_out_p, b_out_p)


def _make_edge_kernel(npad, etot, nch):
    eps = nch * CH
    rows_per = npad // NS
    ZB = 128
    nz = rows_per // ZB

    def body(t_hbm, eaw_hbm, src_hbm, dst_hbm, zero_hbm, out_hbm,
             idx_ag, idx_dr, idx_bg, a_v, b_v, e_v,
             sem_a, sem_b, sem_e, shared):
        c = lax.axis_index("c")
        s = lax.axis_index("s")
        for k in range(nz):
            pltpu.sync_copy(
                zero_hbm, shared.at[pl.ds(s * rows_per + k * ZB, ZB)])
        plsc.subcore_barrier()

        a_off = c * npad
        b_off = (2 + c) * npad
        ebase = c * etot + s * eps

        def chunk_body(j, carry):
            pltpu.sync_copy(src_hbm.at[s, j], idx_ag)
            pltpu.sync_copy(dst_hbm.at[s, j], idx_dr)
            for v in range(CH // 16):
                sl = pl.ds(v * 16, 16)
                idx_bg[sl] = idx_dr[sl] + b_off
                idx_ag[sl] = idx_ag[sl] + a_off
            da = pltpu.async_copy(t_hbm.at[idx_ag], a_v, sem_a)
            db = pltpu.async_copy(t_hbm.at[idx_bg], b_v, sem_b)
            de = pltpu.async_copy(
                eaw_hbm.at[pl.ds(ebase + j * CH, CH)], e_v, sem_e)
            da.wait()
            db.wait()
            de.wait()

            def comp(ei, cc):
                for v in range(LN // 16):
                    sl = pl.ds(v * 16, 16)
                    a_v[ei, sl] = jnp.maximum(
                        a_v[ei, sl] + b_v[ei, sl] + e_v[ei, sl], 0.0)
                return cc

            lax.fori_loop(0, CH, comp, 0)
            pltpu.sync_copy(a_v, shared.at[idx_dr], add=True)
            return carry

        lax.fori_loop(0, nch, chunk_body, 0)
        plsc.subcore_barrier()
        for k in range(nz):
            off = s * rows_per + k * ZB
            pltpu.sync_copy(
                shared.at[pl.ds(off, ZB)],
                out_hbm.at[pl.ds(c * npad + off, ZB)])

    mesh = plsc.VectorSubcoreMesh(
        core_axis_name="c", subcore_axis_name="s",
        num_cores=NC, num_subcores=NS)
    return pl.kernel(
        body,
        out_type=jax.ShapeDtypeStruct((2 * npad, LN), jnp.float32),
        mesh=mesh,
        scratch_types=[
            pltpu.VMEM((CH,), jnp.int32),
            pltpu.VMEM((CH,), jnp.int32),
            pltpu.VMEM((CH,), jnp.int32),
            pltpu.VMEM((CH, LN), jnp.float32),
            pltpu.VMEM((CH, LN), jnp.float32),
            pltpu.VMEM((CH, LN), jnp.float32),
            pltpu.SemaphoreType.DMA,
            pltpu.SemaphoreType.DMA,
            pltpu.SemaphoreType.DMA,
            pltpu.VMEM_SHARED((npad, LN), jnp.float32),
        ],
    )


def _round_up(a, b):
    return -(-a // b) * b


def kernel(x, edge_index, batch, edge_attr, pos, W_msg1, b_msg1, W_upd1,
           b_upd1, W_msg2, b_msg2, W_upd2, b_upd2, W_out, b_out):
    f32 = jnp.float32
    n, f = x.shape
    e = edge_index.shape[1]
    ed = edge_attr.shape[1]
    h = W_upd1.shape[1]
    c_out = W_out.shape[1]

    npad = _round_up(n + 1, NS * 128)
    etot = _round_up(e, NS * CH)
    nch = etot // (NS * CH)
    kp = _round_up(f + 3, 128)

    src = edge_index[0]
    dst = edge_index[1]
    srcp = jnp.concatenate([src, jnp.zeros((etot - e,), jnp.int32)])
    dstp = jnp.concatenate([dst, jnp.full((etot - e,), n, jnp.int32)])
    src3 = srcp.reshape(NS, nch, CH)
    dst3 = dstp.reshape(NS, nch, CH)

    posp = jnp.pad(pos, ((0, npad - n), (0, 0)))
    xpad = jnp.pad(x, ((0, npad - n), (0, 0)))
    zero128 = jnp.zeros((128, LN), f32)

    def node_tables(hh, w_msg, fh):
        wx = w_msg[:fh]
        wp = w_msg[fh + ed:]
        zk = jnp.zeros((kp - fh - 3, w_msg.shape[1]), f32)
        col_a = jnp.concatenate([wx, -wp, zk], axis=0)
        col_b = jnp.concatenate([jnp.zeros((fh, w_msg.shape[1]), f32), wp, zk],
                                axis=0)
        wcat = jnp.concatenate([col_a, col_b], axis=1)
        hp = jnp.concatenate(
            [hh, posp, jnp.zeros((npad, kp - fh - 3), f32)], axis=1)
        return _stacked_matmul(hp, wcat, jnp.zeros((4 * LN,), f32), bm=512)

    eap = jnp.pad(edge_attr, ((0, etot - e), (0, 0)))
    eaw1 = _stacked_matmul(eap, W_msg1[f:f + ed], b_msg1, bm=2016)
    eaw2 = _stacked_matmul(eap, W_msg2[h:h + ed], b_msg2, bm=2016)

    edge_k = _make_edge_kernel(npad, etot, nch)

    wmsg = jnp.stack([W_msg1, W_msg2])
    wupd = jnp.stack([W_upd1, W_upd2])
    bupd = jnp.stack([b_upd1, b_upd2])
    eaws = jnp.stack([eaw1, eaw2])

    def layer_step(hcur, ws):
        wm, wu, bu, eaw_l = ws
        t = node_tables(hcur, wm, f)
        agg = edge_k(t, eaw_l, src3, dst3, zero128)
        hnext = _update_matmul(hcur, agg, wu, bu, bm=512)
        return hnext, 0.0

    h2, _ = lax.scan(layer_step, xpad, (wmsg, wupd, bupd, eaws))

    sums, cnts = _pool(batch, h2, n, bn=400)
    w_out_p = jnp.pad(W_out, ((0, 0), (0, LN - c_out)))
    b_out_p = jnp.tile(b_out.reshape(1, -1), (8, 1))
    b_out_p = jnp.pad(b_out_p, ((0, 0), (0, LN - c_out)))
    out = _head(sums, cnts, w_out_p, b_out_p)
    return out[:, :c_out]


# speedup vs baseline: 1.7535x; 1.2495x over previous
import functools

import jax
import jax.numpy as jnp
from jax import lax
from jax.experimental import pallas as pl
from jax.experimental.pallas import tpu as pltpu
from jax.experimental.pallas import tpu_sc as plsc

G = 64
NC = 2
NS = 16
CH = 48
SK = 14
LN = 128


def _mm_stacked_body(x_ref, w_ref, b_ref, o_ref):
    o_ref[...] = (
        jnp.dot(x_ref[...], w_ref[0], preferred_element_type=jnp.float32)
        + b_ref[0]
    )


def _stacked_matmul(xp, wcat, bcat, bm, ln=LN):
    m, k = xp.shape
    nj = wcat.shape[1] // ln
    ni = m // bm
    w3 = wcat.reshape(k, nj, ln).transpose(1, 0, 2)
    return pl.pallas_call(
        _mm_stacked_body,
        grid=(nj, ni),
        in_specs=[
            pl.BlockSpec((bm, k), lambda j, i: (i, 0)),
            pl.BlockSpec((1, k, ln), lambda j, i: (j, 0, 0)),
            pl.BlockSpec((1, 1, ln), lambda j, i: (j, 0, 0)),
        ],
        out_specs=pl.BlockSpec((bm, ln), lambda j, i, ni=ni: (j * ni + i, 0)),
        out_shape=jax.ShapeDtypeStruct((nj * m, ln), jnp.float32),
    )(xp, w3, bcat.reshape(nj, 1, ln))


def _upd_body(h_ref, alo_ref, ahi_ref, wh_ref, wlo_ref, whi_ref, b_ref,
              o_ref):
    acc = jnp.dot(h_ref[...], wh_ref[...], preferred_element_type=jnp.float32)
    acc += jnp.dot(alo_ref[...], wlo_ref[...],
                   preferred_element_type=jnp.float32)
    acc += jnp.dot(ahi_ref[...], whi_ref[...],
                   preferred_element_type=jnp.float32)
    o_ref[...] = jnp.maximum(acc + b_ref[0], 0.0)


def _update_matmul(h, agg, w_upd, b_upd, bm):
    npad, fh = h.shape
    hout = w_upd.shape[1]
    ni = npad // bm
    nc = hout // LN
    wh = w_upd[:fh]
    wlo = w_upd[fh:fh + LN]
    whi = w_upd[fh + LN:]
    return pl.pallas_call(
        _upd_body,
        grid=(nc, ni),
        in_specs=[
            pl.BlockSpec((bm, fh), lambda c, i: (i, 0)),
            pl.BlockSpec((bm, LN), lambda c, i: (i, 0)),
            pl.BlockSpec((bm, LN), lambda c, i, ni=ni: (ni + i, 0)),
            pl.BlockSpec((fh, LN), lambda c, i: (0, c)),
            pl.BlockSpec((LN, LN), lambda c, i: (0, c)),
            pl.BlockSpec((LN, LN), lambda c, i: (0, c)),
            pl.BlockSpec((1, 1, LN), lambda c, i: (c, 0, 0)),
        ],
        out_specs=pl.BlockSpec((bm, LN), lambda c, i: (i, c)),
        out_shape=jax.ShapeDtypeStruct((npad, hout), jnp.float32),
    )(h, agg, agg, wh, wlo, whi, b_upd.reshape(nc, 1, LN))


def _pool_body(b3_ref, h_ref, sum_ref, cnt_ref):
    i = pl.program_id(0)

    @pl.when(i == 0)
    def _():
        sum_ref[...] = jnp.zeros_like(sum_ref)
        cnt_ref[...] = jnp.zeros_like(cnt_ref)

    bids = b3_ref[0]
    ids = lax.broadcasted_iota(jnp.int32, (G, bids.shape[1]), 0)
    oh = (bids == ids).astype(jnp.float32)
    sum_ref[...] += jnp.dot(oh, h_ref[...], preferred_element_type=jnp.float32)
    cnt_ref[...] += jnp.sum(oh, axis=1, keepdims=True)


def _pool(batch, h, n, bn):
    nb = n // bn
    fh = h.shape[1]
    batch3 = batch.reshape(nb, 1, bn)
    return pl.pallas_call(
        _pool_body,
        grid=(nb,),
        in_specs=[
            pl.BlockSpec((1, 1, bn), lambda i: (i, 0, 0)),
            pl.BlockSpec((bn, fh), lambda i: (i, 0)),
        ],
        out_specs=[
            pl.BlockSpec((G, fh), lambda i: (0, 0)),
            pl.BlockSpec((G, LN), lambda i: (0, 0)),
        ],
        out_shape=[
            jax.ShapeDtypeStruct((G, fh), jnp.float32),
            jax.ShapeDtypeStruct((G, LN), jnp.float32),
        ],
    )(batch3, h)


def _head_body(s_ref, c_ref, w_ref, b_ref, o_ref):
    cnt = c_ref[:, :1]
    pooled = s_ref[...] / jnp.clip(cnt, 1.0, None)
    o_ref[...] = (
        jnp.dot(pooled, w_ref[...], preferred_element_type=jnp.float32)
        + b_ref[0:1, :]
    )


def _head(sums, cnts, w_out_p, b_out_p):
    fh = sums.shape[1]
    return pl.pallas_call(
        _head_body,
        out_shape=jax.ShapeDtypeStruct((G, LN), jnp.float32),
    )(sums, cnts, w_out_p, b_out_p)


def _make_edge_kernel(npad, etot, nch):
    nsc = nch // SK
    eps = nch * CH
    rows_per = npad // NS
    ZB = 128
    nz = rows_per // ZB

    def body(t_hbm, eaw_hbm, src_hbm, dst_hbm, zero_hbm, out_hbm,
             idx_ag, idx_dr, idx_bg,
             a0_v, a1_v, a2_v, b0_v, b1_v, e0_v, e1_v,
             ga0, ga1, ga2, gb0, gb1, ge0, ge1, sc0, sc1, sc2,
             shared):
        a_bufs = (a0_v, a1_v, a2_v)
        b_bufs = (b0_v, b1_v)
        e_bufs = (e0_v, e1_v)
        ga_sem = (ga0, ga1, ga2)
        gb_sem = (gb0, gb1)
        ge_sem = (ge0, ge1)
        sc_sem = (sc0, sc1, sc2)

        c = lax.axis_index("c")
        s = lax.axis_index("s")
        for k in range(nz):
            pltpu.sync_copy(
                zero_hbm, shared.at[pl.ds(s * rows_per + k * ZB, ZB)])
        plsc.subcore_barrier()

        a_off = c * npad
        b_off = (2 + c) * npad
        ebase = c * etot + s * eps

        def super_body(g, carry):
            pltpu.sync_copy(src_hbm.at[s, g], idx_ag)
            pltpu.sync_copy(dst_hbm.at[s, g], idx_dr)

            def off_body(r, cc):
                for v in range(CH // 16):
                    sl = pl.ds(v * 16, 16)
                    idx_bg[r, sl] = idx_dr[r, sl] + b_off
                    idx_ag[r, sl] = idx_ag[r, sl] + a_off
                return cc

            lax.fori_loop(0, SK, off_body, 0)

            gd = {}
            sd = {}
            eg = ebase + g * (SK * CH)

            def gfire(k):
                if k >= 3:
                    sd[k - 3].wait()
                gd[k] = (
                    pltpu.async_copy(
                        t_hbm.at[idx_ag.at[k]], a_bufs[k % 3],
                        ga_sem[k % 3]),
                    pltpu.async_copy(
                        t_hbm.at[idx_bg.at[k]], b_bufs[k % 2],
                        gb_sem[k % 2]),
                    pltpu.async_copy(
                        eaw_hbm.at[pl.ds(eg + k * CH, CH)], e_bufs[k % 2],
                        ge_sem[k % 2]),
                )

            gfire(0)
            gfire(1)
            for k in range(SK):
                a_v, b_v, e_v = a_bufs[k % 3], b_bufs[k % 2], e_bufs[k % 2]
                for d in gd.pop(k):
                    d.wait()

                def comp(ei, cc, a_v=a_v, b_v=b_v, e_v=e_v):
                    for v in range(LN // 16):
                        sl = pl.ds(v * 16, 16)
                        a_v[ei, sl] = jnp.maximum(
                            a_v[ei, sl] + b_v[ei, sl] + e_v[ei, sl], 0.0)
                    return cc

                lax.fori_loop(0, CH, comp, 0)
                sd[k] = pltpu.async_copy(
                    a_v, shared.at[idx_dr.at[k]], sc_sem[k % 3], add=True)
                if k + 2 < SK:
                    gfire(k + 2)
            for k in range(SK - 3, SK):
                sd[k].wait()
            return carry

        lax.fori_loop(0, nsc, super_body, 0)
        plsc.subcore_barrier()
        for k in range(nz):
            off = s * rows_per + k * ZB
            pltpu.sync_copy(
                shared.at[pl.ds(off, ZB)],
                out_hbm.at[pl.ds(c * npad + off, ZB)])

    mesh = plsc.VectorSubcoreMesh(
        core_axis_name="c", subcore_axis_name="s",
        num_cores=NC, num_subcores=NS)
    return pl.kernel(
        body,
        out_type=jax.ShapeDtypeStruct((2 * npad, LN), jnp.float32),
        mesh=mesh,
        scratch_types=[
            pltpu.VMEM((SK, CH), jnp.int32),
            pltpu.VMEM((SK, CH), jnp.int32),
            pltpu.VMEM((SK, CH), jnp.int32),
            pltpu.VMEM((CH, LN), jnp.float32),
            pltpu.VMEM((CH, LN), jnp.float32),
            pltpu.VMEM((CH, LN), jnp.float32),
            pltpu.VMEM((CH, LN), jnp.float32),
            pltpu.VMEM((CH, LN), jnp.float32),
            pltpu.VMEM((CH, LN), jnp.float32),
            pltpu.VMEM((CH, LN), jnp.float32),
            pltpu.SemaphoreType.DMA,
            pltpu.SemaphoreType.DMA,
            pltpu.SemaphoreType.DMA,
            pltpu.SemaphoreType.DMA,
            pltpu.SemaphoreType.DMA,
            pltpu.SemaphoreType.DMA,
            pltpu.SemaphoreType.DMA,
            pltpu.SemaphoreType.DMA,
            pltpu.SemaphoreType.DMA,
            pltpu.SemaphoreType.DMA,
            pltpu.VMEM_SHARED((npad, LN), jnp.float32),
        ],
    )


def _round_up(a, b):
    return -(-a // b) * b


def kernel(x, edge_index, batch, edge_attr, pos, W_msg1, b_msg1, W_upd1,
           b_upd1, W_msg2, b_msg2, W_upd2, b_upd2, W_out, b_out):
    f32 = jnp.float32
    n, f = x.shape
    e = edge_index.shape[1]
    ed = edge_attr.shape[1]
    h = W_upd1.shape[1]
    c_out = W_out.shape[1]

    npad = _round_up(n + 1, NS * 128)
    etot = _round_up(e, NS * CH * SK)
    nch = etot // (NS * CH)
    kp = _round_up(f + 3, 128)

    src = edge_index[0]
    dst = edge_index[1]
    srcp = jnp.concatenate([src, jnp.zeros((etot - e,), jnp.int32)])
    dstp = jnp.concatenate([dst, jnp.full((etot - e,), n, jnp.int32)])
    src3 = srcp.reshape(NS, nch // SK, SK, CH)
    dst3 = dstp.reshape(NS, nch // SK, SK, CH)

    posp = jnp.pad(pos, ((0, npad - n), (0, 0)))
    xpad = jnp.pad(x, ((0, npad - n), (0, 0)))
    zero128 = jnp.zeros((128, LN), f32)

    def node_tables(hh, w_msg, fh):
        wx = w_msg[:fh]
        wp = w_msg[fh + ed:]
        zk = jnp.zeros((kp - fh - 3, w_msg.shape[1]), f32)
        col_a = jnp.concatenate([wx, -wp, zk], axis=0)
        col_b = jnp.concatenate([jnp.zeros((fh, w_msg.shape[1]), f32), wp, zk],
                                axis=0)
        wcat = jnp.concatenate([col_a, col_b], axis=1)
        hp = jnp.concatenate(
            [hh, posp, jnp.zeros((npad, kp - fh - 3), f32)], axis=1)
        return _stacked_matmul(hp, wcat, jnp.zeros((4 * LN,), f32), bm=512)

    eap = jnp.pad(edge_attr, ((0, etot - e), (0, 0)))
    eaw1 = _stacked_matmul(eap, W_msg1[f:f + ed], b_msg1, bm=2016)
    eaw2 = _stacked_matmul(eap, W_msg2[h:h + ed], b_msg2, bm=2016)

    edge_k = _make_edge_kernel(npad, etot, nch)

    wmsg = jnp.stack([W_msg1, W_msg2])
    wupd = jnp.stack([W_upd1, W_upd2])
    bupd = jnp.stack([b_upd1, b_upd2])
    eaws = jnp.stack([eaw1, eaw2])

    def layer_step(hcur, ws):
        wm, wu, bu, eaw_l = ws
        t = node_tables(hcur, wm, f)
        agg = edge_k(t, eaw_l, src3, dst3, zero128)
        hnext = _update_matmul(hcur, agg, wu, bu, bm=512)
        return hnext, 0.0

    h2, _ = lax.scan(layer_step, xpad, (wmsg, wupd, bupd, eaws))

    sums, cnts = _pool(batch, h2, n, bn=400)
    w_out_p = jnp.pad(W_out, ((0, 0), (0, LN - c_out)))
    b_out_p = jnp.tile(b_out.reshape(1, -1), (8, 1))
    b_out_p = jnp.pad(b_out_p, ((0, 0), (0, LN - c_out)))
    out = _head(sums, cnts, w_out_p, b_out_p)
    return out[:, :c_out]


# speedup vs baseline: 2.2535x; 1.2851x over previous
import functools

import jax
import jax.numpy as jnp
from jax import lax
from jax.experimental import pallas as pl
from jax.experimental.pallas import tpu as pltpu
from jax.experimental.pallas import tpu_sc as plsc

G = 64
NC = 2
NS = 16
CH = 48
SK = 10
LN = 128


def _mm_stacked_body(x_ref, w_ref, b_ref, o_ref):
    o_ref[...] = (
        jnp.dot(x_ref[...], w_ref[0], preferred_element_type=jnp.float32)
        + b_ref[0]
    )


def _mm_stacked2_body(x_ref, y_ref, w_ref, u_ref, o_ref):
    o_ref[...] = (
        jnp.dot(x_ref[...], w_ref[0], preferred_element_type=jnp.float32)
        + jnp.dot(y_ref[...], u_ref[0], preferred_element_type=jnp.float32)
    )


def _stacked_matmul2(xp, yp, wcat, ucat, bm, ln=LN):
    m, k1 = xp.shape
    k2 = yp.shape[1]
    nj = wcat.shape[1] // ln
    ni = m // bm
    w3 = wcat.reshape(k1, nj, ln).transpose(1, 0, 2)
    u3 = ucat.reshape(k2, nj, ln).transpose(1, 0, 2)
    return pl.pallas_call(
        _mm_stacked2_body,
        grid=(nj, ni),
        in_specs=[
            pl.BlockSpec((bm, k1), lambda j, i: (i, 0)),
            pl.BlockSpec((bm, k2), lambda j, i: (i, 0)),
            pl.BlockSpec((1, k1, ln), lambda j, i: (j, 0, 0)),
            pl.BlockSpec((1, k2, ln), lambda j, i: (j, 0, 0)),
        ],
        out_specs=pl.BlockSpec((bm, ln), lambda j, i, ni=ni: (j * ni + i, 0)),
        out_shape=jax.ShapeDtypeStruct((nj * m, ln), jnp.float32),
    )(xp, yp, w3, u3)


def _stacked_matmul(xp, wcat, bcat, bm, ln=LN):
    m, k = xp.shape
    nj = wcat.shape[1] // ln
    ni = m // bm
    w3 = wcat.reshape(k, nj, ln).transpose(1, 0, 2)
    return pl.pallas_call(
        _mm_stacked_body,
        grid=(nj, ni),
        in_specs=[
            pl.BlockSpec((bm, k), lambda j, i: (i, 0)),
            pl.BlockSpec((1, k, ln), lambda j, i: (j, 0, 0)),
            pl.BlockSpec((1, 1, ln), lambda j, i: (j, 0, 0)),
        ],
        out_specs=pl.BlockSpec((bm, ln), lambda j, i, ni=ni: (j * ni + i, 0)),
        out_shape=jax.ShapeDtypeStruct((nj * m, ln), jnp.float32),
    )(xp, w3, bcat.reshape(nj, 1, ln))


def _upd_body(h_ref, alo_ref, ahi_ref, wh_ref, wlo_ref, whi_ref, b_ref,
              o_ref):
    acc = jnp.dot(h_ref[...], wh_ref[...], preferred_element_type=jnp.float32)
    acc += jnp.dot(alo_ref[...], wlo_ref[...],
                   preferred_element_type=jnp.float32)
    acc += jnp.dot(ahi_ref[...], whi_ref[...],
                   preferred_element_type=jnp.float32)
    o_ref[...] = jnp.maximum(acc + b_ref[0], 0.0)


def _update_matmul(h, agg, w_upd, b_upd, bm):
    npad, fh = h.shape
    hout = w_upd.shape[1]
    ni = npad // bm
    nc = hout // LN
    wh = w_upd[:fh]
    wlo = w_upd[fh:fh + LN]
    whi = w_upd[fh + LN:]
    return pl.pallas_call(
        _upd_body,
        grid=(nc, ni),
        in_specs=[
            pl.BlockSpec((bm, fh), lambda c, i: (i, 0)),
            pl.BlockSpec((bm, LN), lambda c, i: (i, 0)),
            pl.BlockSpec((bm, LN), lambda c, i, ni=ni: (ni + i, 0)),
            pl.BlockSpec((fh, LN), lambda c, i: (0, c)),
            pl.BlockSpec((LN, LN), lambda c, i: (0, c)),
            pl.BlockSpec((LN, LN), lambda c, i: (0, c)),
            pl.BlockSpec((1, 1, LN), lambda c, i: (c, 0, 0)),
        ],
        out_specs=pl.BlockSpec((bm, LN), lambda c, i: (i, c)),
        out_shape=jax.ShapeDtypeStruct((npad, hout), jnp.float32),
    )(h, agg, agg, wh, wlo, whi, b_upd.reshape(nc, 1, LN))


def _pool_body(b3_ref, h_ref, sum_ref, cnt_ref):
    i = pl.program_id(0)

    @pl.when(i == 0)
    def _():
        sum_ref[...] = jnp.zeros_like(sum_ref)
        cnt_ref[...] = jnp.zeros_like(cnt_ref)

    bids = b3_ref[0]
    ids = lax.broadcasted_iota(jnp.int32, (G, bids.shape[1]), 0)
    oh = (bids == ids).astype(jnp.float32)
    sum_ref[...] += jnp.dot(oh, h_ref[...], preferred_element_type=jnp.float32)
    cnt_ref[...] += jnp.sum(oh, axis=1, keepdims=True)


def _pool(batch, h, n, bn):
    nb = n // bn
    fh = h.shape[1]
    batch3 = batch.reshape(nb, 1, bn)
    return pl.pallas_call(
        _pool_body,
        grid=(nb,),
        in_specs=[
            pl.BlockSpec((1, 1, bn), lambda i: (i, 0, 0)),
            pl.BlockSpec((bn, fh), lambda i: (i, 0)),
        ],
        out_specs=[
            pl.BlockSpec((G, fh), lambda i: (0, 0)),
            pl.BlockSpec((G, LN), lambda i: (0, 0)),
        ],
        out_shape=[
            jax.ShapeDtypeStruct((G, fh), jnp.float32),
            jax.ShapeDtypeStruct((G, LN), jnp.float32),
        ],
    )(batch3, h)


def _head_body(s_ref, c_ref, w_ref, b_ref, o_ref):
    cnt = c_ref[:, :1]
    pooled = s_ref[...] / jnp.clip(cnt, 1.0, None)
    o_ref[...] = (
        jnp.dot(pooled, w_ref[...], preferred_element_type=jnp.float32)
        + b_ref[0:1, :]
    )


def _head(sums, cnts, w_out_p, b_out_p):
    fh = sums.shape[1]
    return pl.pallas_call(
        _head_body,
        out_shape=jax.ShapeDtypeStruct((G, LN), jnp.float32),
    )(sums, cnts, w_out_p, b_out_p)


def _make_edge_kernel(npad, etot, nch):
    nsc = nch // SK
    eps = nch * CH
    rows_per = npad // NS
    ZB = 128
    nz = rows_per // ZB

    def body(t_hbm, eaw_hbm, src_hbm, dst_hbm, zero_hbm, lidx_hbm, out_hbm,
             idx_ag, idx_dr, idx_bg,
             a0_v, a1_v, a2_v, b0_v, b1_v, e0_v, e1_v,
             ga0, ga1, ga2, gb0, gb1, ge0, ge1, sc0, sc1, sc2,
             shared):
        a_bufs = (a0_v, a1_v, a2_v)
        b_bufs = (b0_v, b1_v)
        e_bufs = (e0_v, e1_v)
        ga_sem = (ga0, ga1, ga2)
        gb_sem = (gb0, gb1)
        ge_sem = (ge0, ge1)
        sc_sem = (sc0, sc1, sc2)

        c = lax.axis_index("c")
        s = lax.axis_index("s")
        for k in range(nz):
            pltpu.sync_copy(
                zero_hbm, shared.at[pl.ds(s * rows_per + k * ZB, ZB)])
        plsc.subcore_barrier()

        pltpu.sync_copy(lidx_hbm, idx_ag.at[SK, pl.ds(0, 16)])
        layer = idx_ag[SK, pl.ds(0, 16)][0]
        a_off = c * npad
        b_off = (2 + c) * npad
        ebase = (2 * layer + c) * etot + s * eps

        def super_body(g, carry):
            pltpu.sync_copy(src_hbm.at[s, g], idx_ag.at[pl.ds(0, SK)])
            pltpu.sync_copy(dst_hbm.at[s, g], idx_dr)

            def off_body(r, cc):
                for v in range(CH // 16):
                    sl = pl.ds(v * 16, 16)
                    idx_bg[r, sl] = idx_dr[r, sl] + b_off
                    idx_ag[r, sl] = idx_ag[r, sl] + a_off
                return cc

            lax.fori_loop(0, SK, off_body, 0)

            gd = {}
            sd = {}
            eg = ebase + g * (SK * CH)

            def gfire(k):
                if k >= 3:
                    sd[k - 3].wait()
                gd[k] = (
                    pltpu.async_copy(
                        t_hbm.at[idx_ag.at[k]], a_bufs[k % 3],
                        ga_sem[k % 3]),
                    pltpu.async_copy(
                        t_hbm.at[idx_bg.at[k]], b_bufs[k % 2],
                        gb_sem[k % 2]),
                    pltpu.async_copy(
                        eaw_hbm.at[pl.ds(eg + k * CH, CH)], e_bufs[k % 2],
                        ge_sem[k % 2]),
                )

            gfire(0)
            gfire(1)
            for k in range(SK):
                a_v, b_v, e_v = a_bufs[k % 3], b_bufs[k % 2], e_bufs[k % 2]
                for d in gd.pop(k):
                    d.wait()

                def comp(ei, cc, a_v=a_v, b_v=b_v, e_v=e_v):
                    for v in range(LN // 16):
                        sl = pl.ds(v * 16, 16)
                        a_v[ei, sl] = jnp.maximum(
                            a_v[ei, sl] + b_v[ei, sl] + e_v[ei, sl], 0.0)
                    return cc

                lax.fori_loop(0, CH, comp, 0)
                sd[k] = pltpu.async_copy(
                    a_v, shared.at[idx_dr.at[k]], sc_sem[k % 3], add=True)
                if k + 2 < SK:
                    gfire(k + 2)
            for k in range(SK - 3, SK):
                sd[k].wait()
            return carry

        lax.fori_loop(0, nsc, super_body, 0)
        plsc.subcore_barrier()
        for k in range(nz):
            off = s * rows_per + k * ZB
            pltpu.sync_copy(
                shared.at[pl.ds(off, ZB)],
                out_hbm.at[pl.ds(c * npad + off, ZB)])

    mesh = plsc.VectorSubcoreMesh(
        core_axis_name="c", subcore_axis_name="s",
        num_cores=NC, num_subcores=NS)
    return pl.kernel(
        body,
        out_type=jax.ShapeDtypeStruct((2 * npad, LN), jnp.float32),
        mesh=mesh,
        scratch_types=[
            pltpu.VMEM((SK + 1, CH), jnp.int32),
            pltpu.VMEM((SK, CH), jnp.int32),
            pltpu.VMEM((SK, CH), jnp.int32),
            pltpu.VMEM((CH, LN), jnp.float32),
            pltpu.VMEM((CH, LN), jnp.float32),
            pltpu.VMEM((CH, LN), jnp.float32),
            pltpu.VMEM((CH, LN), jnp.float32),
            pltpu.VMEM((CH, LN), jnp.float32),
            pltpu.VMEM((CH, LN), jnp.float32),
            pltpu.VMEM((CH, LN), jnp.float32),
            pltpu.SemaphoreType.DMA,
            pltpu.SemaphoreType.DMA,
            pltpu.SemaphoreType.DMA,
            pltpu.SemaphoreType.DMA,
            pltpu.SemaphoreType.DMA,
            pltpu.SemaphoreType.DMA,
            pltpu.SemaphoreType.DMA,
            pltpu.SemaphoreType.DMA,
            pltpu.SemaphoreType.DMA,
            pltpu.SemaphoreType.DMA,
            pltpu.VMEM_SHARED((npad, LN), jnp.float32),
        ],
    )


def _round_up(a, b):
    return -(-a // b) * b


def kernel(x, edge_index, batch, edge_attr, pos, W_msg1, b_msg1, W_upd1,
           b_upd1, W_msg2, b_msg2, W_upd2, b_upd2, W_out, b_out):
    f32 = jnp.float32
    n, f = x.shape
    e = edge_index.shape[1]
    ed = edge_attr.shape[1]
    h = W_upd1.shape[1]
    c_out = W_out.shape[1]

    npad = _round_up(n + 1, NS * 128)
    etot = _round_up(e, NS * CH * SK)
    nch = etot // (NS * CH)
    kp = _round_up(f + 3, 128)

    src = edge_index[0]
    dst = edge_index[1]
    srcp = jnp.concatenate([src, jnp.zeros((etot - e,), jnp.int32)])
    dstp = jnp.concatenate([dst, jnp.full((etot - e,), n, jnp.int32)])
    src3 = srcp.reshape(NS, nch // SK, SK, CH)
    dst3 = dstp.reshape(NS, nch // SK, SK, CH)

    posp = jnp.pad(pos, ((0, npad - n), (0, 0)))
    xpad = jnp.pad(x, ((0, npad - n), (0, 0)))
    zero128 = jnp.zeros((128, LN), f32)

    posp8 = jnp.pad(posp, ((0, 0), (0, 5)))

    def node_tables(hh, w_msg, fh):
        hw = w_msg.shape[1]
        wx = w_msg[:fh]
        wp = w_msg[fh + ed:]
        wcat = jnp.concatenate([wx, jnp.zeros((fh, hw), f32)], axis=1)
        wp8 = jnp.pad(jnp.concatenate([-wp, wp], axis=1), ((0, 5), (0, 0)))
        return _stacked_matmul2(hh, posp8, wcat, wp8, bm=512)

    eap = jnp.pad(edge_attr, ((0, etot - e), (0, 0)))
    wecat = jnp.concatenate([W_msg1[f:f + ed], W_msg2[h:h + ed]], axis=1)
    becat = jnp.concatenate([b_msg1, b_msg2])
    eaw_both = _stacked_matmul(eap, wecat, becat, bm=2016)

    edge_k = _make_edge_kernel(npad, etot, nch)

    wmsg = jnp.stack([W_msg1, W_msg2])
    wupd = jnp.stack([W_upd1, W_upd2])
    bupd = jnp.stack([b_upd1, b_upd2])
    lidx = jnp.stack([jnp.zeros((16,), jnp.int32),
                      jnp.ones((16,), jnp.int32)])

    def layer_step(hcur, ws):
        wm, wu, bu, li = ws
        t = node_tables(hcur, wm, f)
        agg = edge_k(t, eaw_both, src3, dst3, zero128, li)
        hnext = _update_matmul(hcur, agg, wu, bu, bm=512)
        return hnext, 0.0

    h2, _ = lax.scan(layer_step, xpad, (wmsg, wupd, bupd, lidx))

    sums, cnts = _pool(batch, h2, n, bn=400)
    w_out_p = jnp.pad(W_out, ((0, 0), (0, LN - c_out)))
    b_out_p = jnp.tile(b_out.reshape(1, -1), (8, 1))
    b_out_p = jnp.pad(b_out_p, ((0, 0), (0, LN - c_out)))
    out = _head(sums, cnts, w_out_p, b_out_p)
    return out[:, :c_out]


# speedup vs baseline: 2.4505x; 1.0874x over previous
import functools

import jax
import jax.numpy as jnp
from jax import lax
from jax.experimental import pallas as pl
from jax.experimental.pallas import tpu as pltpu
from jax.experimental.pallas import tpu_sc as plsc

G = 64
NC = 2
NS = 16
CH = 48
SK = 10
LN = 128


def _mm_stacked_body(x_ref, w_ref, b_ref, o_ref):
    o_ref[...] = (
        jnp.dot(x_ref[...], w_ref[0], preferred_element_type=jnp.float32)
        + b_ref[0]
    )


def _mm_stackedT_body(xt_ref, w_ref, b_ref, o_ref):
    o_ref[...] = (
        lax.dot_general(xt_ref[...], w_ref[0], (((0,), (0,)), ((), ())),
                        preferred_element_type=jnp.float32)
        + b_ref[0]
    )


def _stacked_matmulT(xt, wcat, bcat, bm, ln=LN):
    k, m = xt.shape
    nj = wcat.shape[1] // ln
    ni = m // bm
    w3 = wcat.reshape(k, nj, ln).transpose(1, 0, 2)
    return pl.pallas_call(
        _mm_stackedT_body,
        grid=(nj, ni),
        in_specs=[
            pl.BlockSpec((k, bm), lambda j, i: (0, i)),
            pl.BlockSpec((1, k, ln), lambda j, i: (j, 0, 0)),
            pl.BlockSpec((1, 1, ln), lambda j, i: (j, 0, 0)),
        ],
        out_specs=pl.BlockSpec((bm, ln), lambda j, i, ni=ni: (j * ni + i, 0)),
        out_shape=jax.ShapeDtypeStruct((nj * m, ln), jnp.float32),
    )(xt, w3, bcat.reshape(nj, 1, ln))


def _mm_stacked2_body(x_ref, y_ref, w_ref, u_ref, o_ref):
    o_ref[...] = (
        jnp.dot(x_ref[...], w_ref[0], preferred_element_type=jnp.float32)
        + jnp.dot(y_ref[...], u_ref[0], preferred_element_type=jnp.float32)
    )


def _stacked_matmul2(xp, yp, wcat, ucat, bm, ln=LN):
    m, k1 = xp.shape
    k2 = yp.shape[1]
    nj = wcat.shape[1] // ln
    ni = m // bm
    w3 = wcat.reshape(k1, nj, ln).transpose(1, 0, 2)
    u3 = ucat.reshape(k2, nj, ln).transpose(1, 0, 2)
    return pl.pallas_call(
        _mm_stacked2_body,
        grid=(nj, ni),
        in_specs=[
            pl.BlockSpec((bm, k1), lambda j, i: (i, 0)),
            pl.BlockSpec((bm, k2), lambda j, i: (i, 0)),
            pl.BlockSpec((1, k1, ln), lambda j, i: (j, 0, 0)),
            pl.BlockSpec((1, k2, ln), lambda j, i: (j, 0, 0)),
        ],
        out_specs=pl.BlockSpec((bm, ln), lambda j, i, ni=ni: (j * ni + i, 0)),
        out_shape=jax.ShapeDtypeStruct((nj * m, ln), jnp.float32),
    )(xp, yp, w3, u3)


def _stacked_matmul(xp, wcat, bcat, bm, ln=LN):
    m, k = xp.shape
    nj = wcat.shape[1] // ln
    ni = m // bm
    w3 = wcat.reshape(k, nj, ln).transpose(1, 0, 2)
    return pl.pallas_call(
        _mm_stacked_body,
        grid=(nj, ni),
        in_specs=[
            pl.BlockSpec((bm, k), lambda j, i: (i, 0)),
            pl.BlockSpec((1, k, ln), lambda j, i: (j, 0, 0)),
            pl.BlockSpec((1, 1, ln), lambda j, i: (j, 0, 0)),
        ],
        out_specs=pl.BlockSpec((bm, ln), lambda j, i, ni=ni: (j * ni + i, 0)),
        out_shape=jax.ShapeDtypeStruct((nj * m, ln), jnp.float32),
    )(xp, w3, bcat.reshape(nj, 1, ln))


def _upd_body(h_ref, alo_ref, ahi_ref, wh_ref, wlo_ref, whi_ref, b_ref,
              o_ref):
    acc = jnp.dot(h_ref[...], wh_ref[...], preferred_element_type=jnp.float32)
    acc += jnp.dot(alo_ref[...], wlo_ref[...],
                   preferred_element_type=jnp.float32)
    acc += jnp.dot(ahi_ref[...], whi_ref[...],
                   preferred_element_type=jnp.float32)
    o_ref[...] = jnp.maximum(acc + b_ref[0], 0.0)


def _update_matmul(h, agg, w_upd, b_upd, bm):
    npad, fh = h.shape
    hout = w_upd.shape[1]
    ni = npad // bm
    nc = hout // LN
    wh = w_upd[:fh]
    wlo = w_upd[fh:fh + LN]
    whi = w_upd[fh + LN:]
    return pl.pallas_call(
        _upd_body,
        grid=(nc, ni),
        in_specs=[
            pl.BlockSpec((bm, fh), lambda c, i: (i, 0)),
            pl.BlockSpec((bm, LN), lambda c, i: (i, 0)),
            pl.BlockSpec((bm, LN), lambda c, i, ni=ni: (ni + i, 0)),
            pl.BlockSpec((fh, LN), lambda c, i: (0, c)),
            pl.BlockSpec((LN, LN), lambda c, i: (0, c)),
            pl.BlockSpec((LN, LN), lambda c, i: (0, c)),
            pl.BlockSpec((1, 1, LN), lambda c, i: (c, 0, 0)),
        ],
        out_specs=pl.BlockSpec((bm, LN), lambda c, i: (i, c)),
        out_shape=jax.ShapeDtypeStruct((npad, hout), jnp.float32),
    )(h, agg, agg, wh, wlo, whi, b_upd.reshape(nc, 1, LN))


def _pool_body(b3_ref, h_ref, sum_ref, cnt_ref):
    i = pl.program_id(0)

    @pl.when(i == 0)
    def _():
        sum_ref[...] = jnp.zeros_like(sum_ref)
        cnt_ref[...] = jnp.zeros_like(cnt_ref)

    bids = b3_ref[0]
    ids = lax.broadcasted_iota(jnp.int32, (G, bids.shape[1]), 0)
    oh = (bids == ids).astype(jnp.float32)
    sum_ref[...] += jnp.dot(oh, h_ref[...], preferred_element_type=jnp.float32)
    cnt_ref[...] += jnp.sum(oh, axis=1, keepdims=True)


def _pool(batch, h, n, bn):
    nb = n // bn
    fh = h.shape[1]
    batch3 = batch.reshape(nb, 1, bn)
    return pl.pallas_call(
        _pool_body,
        grid=(nb,),
        in_specs=[
            pl.BlockSpec((1, 1, bn), lambda i: (i, 0, 0)),
            pl.BlockSpec((bn, fh), lambda i: (i, 0)),
        ],
        out_specs=[
            pl.BlockSpec((G, fh), lambda i: (0, 0)),
            pl.BlockSpec((G, LN), lambda i: (0, 0)),
        ],
        out_shape=[
            jax.ShapeDtypeStruct((G, fh), jnp.float32),
            jax.ShapeDtypeStruct((G, LN), jnp.float32),
        ],
    )(batch3, h)


def _head_body(s_ref, c_ref, w_ref, b_ref, o_ref):
    cnt = c_ref[:, :1]
    pooled = s_ref[...] / jnp.clip(cnt, 1.0, None)
    o_ref[...] = (
        jnp.dot(pooled, w_ref[...], preferred_element_type=jnp.float32)
        + b_ref[0:1, :]
    )


def _head(sums, cnts, w_out_p, b_out_p):
    fh = sums.shape[1]
    return pl.pallas_call(
        _head_body,
        out_shape=jax.ShapeDtypeStruct((G, LN), jnp.float32),
    )(sums, cnts, w_out_p, b_out_p)


def _make_edge_kernel(npad, etot, nch):
    nsc = nch // SK
    eps = nch * CH
    rows_per = npad // NS
    ZB = 128
    nz = rows_per // ZB

    def body(t_hbm, eaw_hbm, src_hbm, dst_hbm, zero_hbm, lidx_hbm, out_hbm,
             idx_ag, idx_dr, idx_bg,
             a0_v, a1_v, a2_v, b0_v, b1_v, e0_v, e1_v,
             ga0, ga1, ga2, gb0, gb1, ge0, ge1, sc0, sc1, sc2,
             shared):
        a_bufs = (a0_v, a1_v, a2_v)
        b_bufs = (b0_v, b1_v)
        e_bufs = (e0_v, e1_v)
        ga_sem = (ga0, ga1, ga2)
        gb_sem = (gb0, gb1)
        ge_sem = (ge0, ge1)
        sc_sem = (sc0, sc1, sc2)

        c = lax.axis_index("c")
        s = lax.axis_index("s")
        for k in range(nz):
            pltpu.sync_copy(
                zero_hbm, shared.at[pl.ds(s * rows_per + k * ZB, ZB)])
        plsc.subcore_barrier()

        pltpu.sync_copy(lidx_hbm, idx_ag.at[SK, pl.ds(0, 16)])
        layer = idx_ag[SK, pl.ds(0, 16)][0]
        a_off = c * npad
        b_off = (2 + c) * npad
        ebase = (2 * layer + c) * etot + s * eps

        def super_body(g, carry):
            pltpu.sync_copy(src_hbm.at[s, g], idx_ag.at[pl.ds(0, SK)])
            pltpu.sync_copy(dst_hbm.at[s, g], idx_dr)

            def off_body(r, cc):
                for v in range(CH // 16):
                    sl = pl.ds(v * 16, 16)
                    idx_bg[r, sl] = idx_dr[r, sl] + b_off
                    idx_ag[r, sl] = idx_ag[r, sl] + a_off
                return cc

            lax.fori_loop(0, SK, off_body, 0)

            gd = {}
            sd = {}
            eg = ebase + g * (SK * CH)

            def gfire(k):
                if k >= 3:
                    sd[k - 3].wait()
                gd[k] = (
                    pltpu.async_copy(
                        t_hbm.at[idx_ag.at[k]], a_bufs[k % 3],
                        ga_sem[k % 3]),
                    pltpu.async_copy(
                        t_hbm.at[idx_bg.at[k]], b_bufs[k % 2],
                        gb_sem[k % 2]),
                    pltpu.async_copy(
                        eaw_hbm.at[pl.ds(eg + k * CH, CH)], e_bufs[k % 2],
                        ge_sem[k % 2]),
                )

            gfire(0)
            gfire(1)
            for k in range(SK):
                a_v, b_v, e_v = a_bufs[k % 3], b_bufs[k % 2], e_bufs[k % 2]
                for d in gd.pop(k):
                    d.wait()

                def comp(ei, cc, a_v=a_v, b_v=b_v, e_v=e_v):
                    for v in range(LN // 16):
                        sl = pl.ds(v * 16, 16)
                        a_v[ei, sl] = jnp.maximum(
                            a_v[ei, sl] + b_v[ei, sl] + e_v[ei, sl], 0.0)
                    return cc

                lax.fori_loop(0, CH, comp, 0)
                sd[k] = pltpu.async_copy(
                    a_v, shared.at[idx_dr.at[k]], sc_sem[k % 3], add=True)
                if k + 2 < SK:
                    gfire(k + 2)
            for k in range(SK - 3, SK):
                sd[k].wait()
            return carry

        lax.fori_loop(0, nsc, super_body, 0)
        plsc.subcore_barrier()
        for k in range(nz):
            off = s * rows_per + k * ZB
            pltpu.sync_copy(
                shared.at[pl.ds(off, ZB)],
                out_hbm.at[pl.ds(c * npad + off, ZB)])

    mesh = plsc.VectorSubcoreMesh(
        core_axis_name="c", subcore_axis_name="s",
        num_cores=NC, num_subcores=NS)
    return pl.kernel(
        body,
        out_type=jax.ShapeDtypeStruct((2 * npad, LN), jnp.float32),
        mesh=mesh,
        scratch_types=[
            pltpu.VMEM((SK + 1, CH), jnp.int32),
            pltpu.VMEM((SK, CH), jnp.int32),
            pltpu.VMEM((SK, CH), jnp.int32),
            pltpu.VMEM((CH, LN), jnp.float32),
            pltpu.VMEM((CH, LN), jnp.float32),
            pltpu.VMEM((CH, LN), jnp.float32),
            pltpu.VMEM((CH, LN), jnp.float32),
            pltpu.VMEM((CH, LN), jnp.float32),
            pltpu.VMEM((CH, LN), jnp.float32),
            pltpu.VMEM((CH, LN), jnp.float32),
            pltpu.SemaphoreType.DMA,
            pltpu.SemaphoreType.DMA,
            pltpu.SemaphoreType.DMA,
            pltpu.SemaphoreType.DMA,
            pltpu.SemaphoreType.DMA,
            pltpu.SemaphoreType.DMA,
            pltpu.SemaphoreType.DMA,
            pltpu.SemaphoreType.DMA,
            pltpu.SemaphoreType.DMA,
            pltpu.SemaphoreType.DMA,
            pltpu.VMEM_SHARED((npad, LN), jnp.float32),
        ],
    )


def _round_up(a, b):
    return -(-a // b) * b


def kernel(x, edge_index, batch, edge_attr, pos, W_msg1, b_msg1, W_upd1,
           b_upd1, W_msg2, b_msg2, W_upd2, b_upd2, W_out, b_out):
    f32 = jnp.float32
    n, f = x.shape
    e = edge_index.shape[1]
    ed = edge_attr.shape[1]
    h = W_upd1.shape[1]
    c_out = W_out.shape[1]

    npad = _round_up(n + 1, NS * 128)
    etot = _round_up(e, NS * CH * SK)
    nch = etot // (NS * CH)
    kp = _round_up(f + 3, 128)

    src = edge_index[0]
    dst = edge_index[1]
    srcp = jnp.concatenate([src, jnp.zeros((etot - e,), jnp.int32)])
    dstp = jnp.concatenate([dst, jnp.full((etot - e,), n, jnp.int32)])
    src3 = srcp.reshape(NS, nch // SK, SK, CH)
    dst3 = dstp.reshape(NS, nch // SK, SK, CH)

    posp = jnp.pad(pos, ((0, npad - n), (0, 0)))
    xpad = jnp.pad(x, ((0, npad - n), (0, 0)))
    zero128 = jnp.zeros((128, LN), f32)

    posp8 = jnp.pad(posp, ((0, 0), (0, 5)))

    def node_tables(hh, w_msg, fh):
        hw = w_msg.shape[1]
        wx = w_msg[:fh]
        wp = w_msg[fh + ed:]
        wcat = jnp.concatenate([wx, jnp.zeros((fh, hw), f32)], axis=1)
        wp8 = jnp.pad(jnp.concatenate([-wp, wp], axis=1), ((0, 5), (0, 0)))
        return _stacked_matmul2(hh, posp8, wcat, wp8, bm=512)

    eat = jnp.pad(edge_attr.T, ((0, 0), (0, etot - e)))
    wecat = jnp.concatenate([W_msg1[f:f + ed], W_msg2[h:h + ed]], axis=1)
    becat = jnp.concatenate([b_msg1, b_msg2])
    eaw_both = _stacked_matmulT(eat, wecat, becat, bm=2560)

    edge_k = _make_edge_kernel(npad, etot, nch)

    wmsg = jnp.stack([W_msg1, W_msg2])
    wupd = jnp.stack([W_upd1, W_upd2])
    bupd = jnp.stack([b_upd1, b_upd2])
    lidx = jnp.stack([jnp.zeros((16,), jnp.int32),
                      jnp.ones((16,), jnp.int32)])

    def layer_step(hcur, ws):
        wm, wu, bu, li = ws
        t = node_tables(hcur, wm, f)
        agg = edge_k(t, eaw_both, src3, dst3, zero128, li)
        hnext = _update_matmul(hcur, agg, wu, bu, bm=512)
        return hnext, 0.0

    h2, _ = lax.scan(layer_step, xpad, (wmsg, wupd, bupd, lidx))

    sums, cnts = _pool(batch, h2, n, bn=400)
    w_out_p = jnp.pad(W_out, ((0, 0), (0, LN - c_out)))
    b_out_p = jnp.tile(b_out.reshape(1, -1), (8, 1))
    b_out_p = jnp.pad(b_out_p, ((0, 0), (0, LN - c_out)))
    out = _head(sums, cnts, w_out_p, b_out_p)
    return out[:, :c_out]


# speedup vs baseline: 2.5723x; 1.0497x over previous
import functools

import jax
import jax.numpy as jnp
from jax import lax
from jax.experimental import pallas as pl
from jax.experimental.pallas import tpu as pltpu
from jax.experimental.pallas import tpu_sc as plsc

G = 64
NC = 2
NS = 16
CH = 48
SK = 15
LN = 128


def _mm_stacked_body(x_ref, w_ref, b_ref, o_ref):
    o_ref[...] = (
        jnp.dot(x_ref[...], w_ref[0], preferred_element_type=jnp.float32)
        + b_ref[0]
    )


def _mm_stackedT_body(xt_ref, w_ref, b_ref, o_ref):
    o_ref[...] = (
        lax.dot_general(xt_ref[...], w_ref[0], (((0,), (0,)), ((), ())),
                        preferred_element_type=jnp.float32)
        + b_ref[0]
    )


def _stacked_matmulT(xt, wcat, bcat, bm, ln=LN):
    k, m = xt.shape
    nj = wcat.shape[1] // ln
    ni = m // bm
    w3 = wcat.reshape(k, nj, ln).transpose(1, 0, 2)
    return pl.pallas_call(
        _mm_stackedT_body,
        grid=(nj, ni),
        in_specs=[
            pl.BlockSpec((k, bm), lambda j, i: (0, i)),
            pl.BlockSpec((1, k, ln), lambda j, i: (j, 0, 0)),
            pl.BlockSpec((1, 1, ln), lambda j, i: (j, 0, 0)),
        ],
        out_specs=pl.BlockSpec((bm, ln), lambda j, i, ni=ni: (j * ni + i, 0)),
        out_shape=jax.ShapeDtypeStruct((nj * m, ln), jnp.float32),
    )(xt, w3, bcat.reshape(nj, 1, ln))


def _mm_stacked2_body(x_ref, y_ref, w_ref, u_ref, o_ref):
    o_ref[...] = (
        jnp.dot(x_ref[...], w_ref[0], preferred_element_type=jnp.float32)
        + jnp.dot(y_ref[...], u_ref[0], preferred_element_type=jnp.float32)
    )


def _stacked_matmul2(xp, yp, wcat, ucat, bm, ln=LN):
    m, k1 = xp.shape
    k2 = yp.shape[1]
    nj = wcat.shape[1] // ln
    ni = m // bm
    w3 = wcat.reshape(k1, nj, ln).transpose(1, 0, 2)
    u3 = ucat.reshape(k2, nj, ln).transpose(1, 0, 2)
    return pl.pallas_call(
        _mm_stacked2_body,
        grid=(nj, ni),
        in_specs=[
            pl.BlockSpec((bm, k1), lambda j, i: (i, 0)),
            pl.BlockSpec((bm, k2), lambda j, i: (i, 0)),
            pl.BlockSpec((1, k1, ln), lambda j, i: (j, 0, 0)),
            pl.BlockSpec((1, k2, ln), lambda j, i: (j, 0, 0)),
        ],
        out_specs=pl.BlockSpec((bm, ln), lambda j, i, ni=ni: (j * ni + i, 0)),
        out_shape=jax.ShapeDtypeStruct((nj * m, ln), jnp.float32),
    )(xp, yp, w3, u3)


def _stacked_matmul(xp, wcat, bcat, bm, ln=LN):
    m, k = xp.shape
    nj = wcat.shape[1] // ln
    ni = m // bm
    w3 = wcat.reshape(k, nj, ln).transpose(1, 0, 2)
    return pl.pallas_call(
        _mm_stacked_body,
        grid=(nj, ni),
        in_specs=[
            pl.BlockSpec((bm, k), lambda j, i: (i, 0)),
            pl.BlockSpec((1, k, ln), lambda j, i: (j, 0, 0)),
            pl.BlockSpec((1, 1, ln), lambda j, i: (j, 0, 0)),
        ],
        out_specs=pl.BlockSpec((bm, ln), lambda j, i, ni=ni: (j * ni + i, 0)),
        out_shape=jax.ShapeDtypeStruct((nj * m, ln), jnp.float32),
    )(xp, w3, bcat.reshape(nj, 1, ln))


def _upd_body(h_ref, alo_ref, ahi_ref, wh_ref, wlo_ref, whi_ref, b_ref,
              o_ref):
    acc = jnp.dot(h_ref[...], wh_ref[...], preferred_element_type=jnp.float32)
    acc += jnp.dot(alo_ref[...], wlo_ref[...],
                   preferred_element_type=jnp.float32)
    acc += jnp.dot(ahi_ref[...], whi_ref[...],
                   preferred_element_type=jnp.float32)
    o_ref[...] = jnp.maximum(acc + b_ref[0], 0.0)


def _update_matmul(h, agg, w_upd, b_upd, bm):
    npad, fh = h.shape
    hout = w_upd.shape[1]
    ni = npad // bm
    nc = hout // LN
    wh = w_upd[:fh]
    wlo = w_upd[fh:fh + LN]
    whi = w_upd[fh + LN:]
    return pl.pallas_call(
        _upd_body,
        grid=(nc, ni),
        in_specs=[
            pl.BlockSpec((bm, fh), lambda c, i: (i, 0)),
            pl.BlockSpec((bm, LN), lambda c, i: (i, 0)),
            pl.BlockSpec((bm, LN), lambda c, i, ni=ni: (ni + i, 0)),
            pl.BlockSpec((fh, LN), lambda c, i: (0, c)),
            pl.BlockSpec((LN, LN), lambda c, i: (0, c)),
            pl.BlockSpec((LN, LN), lambda c, i: (0, c)),
            pl.BlockSpec((1, 1, LN), lambda c, i: (c, 0, 0)),
        ],
        out_specs=pl.BlockSpec((bm, LN), lambda c, i: (i, c)),
        out_shape=jax.ShapeDtypeStruct((npad, hout), jnp.float32),
    )(h, agg, agg, wh, wlo, whi, b_upd.reshape(nc, 1, LN))


def _pool_body(b3_ref, h_ref, sum_ref, cnt_ref):
    i = pl.program_id(0)

    @pl.when(i == 0)
    def _():
        sum_ref[...] = jnp.zeros_like(sum_ref)
        cnt_ref[...] = jnp.zeros_like(cnt_ref)

    bids = b3_ref[0]
    ids = lax.broadcasted_iota(jnp.int32, (G, bids.shape[1]), 0)
    oh = (bids == ids).astype(jnp.float32)
    sum_ref[...] += jnp.dot(oh, h_ref[...], preferred_element_type=jnp.float32)
    cnt_ref[...] += jnp.sum(oh, axis=1, keepdims=True)


def _pool(batch, h, n, bn):
    nb = n // bn
    fh = h.shape[1]
    batch3 = batch.reshape(nb, 1, bn)
    return pl.pallas_call(
        _pool_body,
        grid=(nb,),
        in_specs=[
            pl.BlockSpec((1, 1, bn), lambda i: (i, 0, 0)),
            pl.BlockSpec((bn, fh), lambda i: (i, 0)),
        ],
        out_specs=[
            pl.BlockSpec((G, fh), lambda i: (0, 0)),
            pl.BlockSpec((G, LN), lambda i: (0, 0)),
        ],
        out_shape=[
            jax.ShapeDtypeStruct((G, fh), jnp.float32),
            jax.ShapeDtypeStruct((G, LN), jnp.float32),
        ],
    )(batch3, h)


def _head_body(s_ref, c_ref, w_ref, b_ref, o_ref):
    cnt = c_ref[:, :1]
    pooled = s_ref[...] / jnp.clip(cnt, 1.0, None)
    o_ref[...] = (
        jnp.dot(pooled, w_ref[...], preferred_element_type=jnp.float32)
        + b_ref[0:1, :]
    )


def _head(sums, cnts, w_out_p, b_out_p):
    fh = sums.shape[1]
    return pl.pallas_call(
        _head_body,
        out_shape=jax.ShapeDtypeStruct((G, LN), jnp.float32),
    )(sums, cnts, w_out_p, b_out_p)


def _make_edge_kernel(npad, etot, nch):
    nsc = nch // SK
    eps = nch * CH
    rows_per = npad // NS
    ZB = 128
    nz = rows_per // ZB

    def body(t_hbm, eaw_hbm, src_hbm, dst_hbm, zero_hbm, lidx_hbm, out_hbm,
             idx_ag, idx_dr, idx_bg,
             a0_v, a1_v, a2_v, b0_v, b1_v, e0_v, e1_v,
             ga0, ga1, ga2, gb0, gb1, ge0, ge1, sc0, sc1, sc2,
             shared):
        a_bufs = (a0_v, a1_v, a2_v)
        b_bufs = (b0_v, b1_v)
        e_bufs = (e0_v, e1_v)
        ga_sem = (ga0, ga1, ga2)
        gb_sem = (gb0, gb1)
        ge_sem = (ge0, ge1)
        sc_sem = (sc0, sc1, sc2)

        c = lax.axis_index("c")
        s = lax.axis_index("s")
        for k in range(nz):
            pltpu.sync_copy(
                zero_hbm, shared.at[pl.ds(s * rows_per + k * ZB, ZB)])
        plsc.subcore_barrier()

        pltpu.sync_copy(lidx_hbm, idx_ag.at[SK, pl.ds(0, 16)])
        layer = idx_ag[SK, pl.ds(0, 16)][0]
        a_off = c * npad
        b_off = (2 + c) * npad
        ebase = (2 * layer + c) * etot + s * eps

        def super_body(g, carry):
            pltpu.sync_copy(src_hbm.at[s, g], idx_ag.at[pl.ds(0, SK)])
            pltpu.sync_copy(dst_hbm.at[s, g], idx_dr)

            def off_body(r, cc):
                for v in range(CH // 16):
                    sl = pl.ds(v * 16, 16)
                    idx_bg[r, sl] = idx_dr[r, sl] + b_off
                    idx_ag[r, sl] = idx_ag[r, sl] + a_off
                return cc

            lax.fori_loop(0, SK, off_body, 0)

            gd = {}
            sd = {}
            eg = ebase + g * (SK * CH)

            def gfire(k):
                if k >= 3:
                    sd[k - 3].wait()
                gd[k] = (
                    pltpu.async_copy(
                        t_hbm.at[idx_ag.at[k]], a_bufs[k % 3],
                        ga_sem[k % 3]),
                    pltpu.async_copy(
                        t_hbm.at[idx_bg.at[k]], b_bufs[k % 2],
                        gb_sem[k % 2]),
                    pltpu.async_copy(
                        eaw_hbm.at[pl.ds(eg + k * CH, CH)], e_bufs[k % 2],
                        ge_sem[k % 2]),
                )

            gfire(0)
            gfire(1)
            for k in range(SK):
                a_v, b_v, e_v = a_bufs[k % 3], b_bufs[k % 2], e_bufs[k % 2]
                for d in gd.pop(k):
                    d.wait()

                def comp(ei, cc, a_v=a_v, b_v=b_v, e_v=e_v):
                    for v in range(LN // 16):
                        sl = pl.ds(v * 16, 16)
                        a_v[ei, sl] = jnp.maximum(
                            a_v[ei, sl] + b_v[ei, sl] + e_v[ei, sl], 0.0)
                    return cc

                lax.fori_loop(0, CH, comp, 0)
                sd[k] = pltpu.async_copy(
                    a_v, shared.at[idx_dr.at[k]], sc_sem[k % 3], add=True)
                if k + 2 < SK:
                    gfire(k + 2)
            for k in range(SK - 3, SK):
                sd[k].wait()
            return carry

        lax.fori_loop(0, nsc, super_body, 0)
        plsc.subcore_barrier()
        for k in range(nz):
            off = s * rows_per + k * ZB
            pltpu.sync_copy(
                shared.at[pl.ds(off, ZB)],
                out_hbm.at[pl.ds(c * npad + off, ZB)])

    mesh = plsc.VectorSubcoreMesh(
        core_axis_name="c", subcore_axis_name="s",
        num_cores=NC, num_subcores=NS)
    return pl.kernel(
        body,
        out_type=jax.ShapeDtypeStruct((2 * npad, LN), jnp.float32),
        mesh=mesh,
        scratch_types=[
            pltpu.VMEM((SK + 1, CH), jnp.int32),
            pltpu.VMEM((SK, CH), jnp.int32),
            pltpu.VMEM((SK, CH), jnp.int32),
            pltpu.VMEM((CH, LN), jnp.float32),
            pltpu.VMEM((CH, LN), jnp.float32),
            pltpu.VMEM((CH, LN), jnp.float32),
            pltpu.VMEM((CH, LN), jnp.float32),
            pltpu.VMEM((CH, LN), jnp.float32),
            pltpu.VMEM((CH, LN), jnp.float32),
            pltpu.VMEM((CH, LN), jnp.float32),
            pltpu.SemaphoreType.DMA,
            pltpu.SemaphoreType.DMA,
            pltpu.SemaphoreType.DMA,
            pltpu.SemaphoreType.DMA,
            pltpu.SemaphoreType.DMA,
            pltpu.SemaphoreType.DMA,
            pltpu.SemaphoreType.DMA,
            pltpu.SemaphoreType.DMA,
            pltpu.SemaphoreType.DMA,
            pltpu.SemaphoreType.DMA,
            pltpu.VMEM_SHARED((npad, LN), jnp.float32),
        ],
    )


def _round_up(a, b):
    return -(-a // b) * b


def kernel(x, edge_index, batch, edge_attr, pos, W_msg1, b_msg1, W_upd1,
           b_upd1, W_msg2, b_msg2, W_upd2, b_upd2, W_out, b_out):
    f32 = jnp.float32
    n, f = x.shape
    e = edge_index.shape[1]
    ed = edge_attr.shape[1]
    h = W_upd1.shape[1]
    c_out = W_out.shape[1]

    npad = _round_up(n + 1, NS * 128)
    etot = _round_up(e, NS * CH * SK)
    nch = etot // (NS * CH)
    kp = _round_up(f + 3, 128)

    src = edge_index[0]
    dst = edge_index[1]
    srcp = jnp.concatenate([src, jnp.zeros((etot - e,), jnp.int32)])
    dstp = jnp.concatenate([dst, jnp.full((etot - e,), n, jnp.int32)])
    src3 = srcp.reshape(NS, nch // SK, SK, CH)
    dst3 = dstp.reshape(NS, nch // SK, SK, CH)

    posp = jnp.pad(pos, ((0, npad - n), (0, 0)))
    xpad = jnp.pad(x, ((0, npad - n), (0, 0)))
    zero128 = jnp.zeros((128, LN), f32)

    posp8 = jnp.pad(posp, ((0, 0), (0, 5)))

    def node_tables(hh, w_msg, fh):
        hw = w_msg.shape[1]
        wx = w_msg[:fh]
        wp = w_msg[fh + ed:]
        wcat = jnp.concatenate([wx, jnp.zeros((fh, hw), f32)], axis=1)
        wp8 = jnp.pad(jnp.concatenate([-wp, wp], axis=1), ((0, 5), (0, 0)))
        return _stacked_matmul2(hh, posp8, wcat, wp8, bm=512)

    eat = jnp.pad(edge_attr.T, ((0, 0), (0, etot - e)))
    wecat = jnp.concatenate([W_msg1[f:f + ed], W_msg2[h:h + ed]], axis=1)
    becat = jnp.concatenate([b_msg1, b_msg2])
    eaw_both = _stacked_matmulT(eat, wecat, becat, bm=2560)

    edge_k = _make_edge_kernel(npad, etot, nch)

    wmsg = jnp.stack([W_msg1, W_msg2])
    wupd = jnp.stack([W_upd1, W_upd2])
    bupd = jnp.stack([b_upd1, b_upd2])
    lidx = jnp.stack([jnp.zeros((16,), jnp.int32),
                      jnp.ones((16,), jnp.int32)])

    def layer_step(hcur, ws):
        wm, wu, bu, li = ws
        t = node_tables(hcur, wm, f)
        agg = edge_k(t, eaw_both, src3, dst3, zero128, li)
        hnext = _update_matmul(hcur, agg, wu, bu, bm=512)
        return hnext, 0.0

    h2, _ = lax.scan(layer_step, xpad, (wmsg, wupd, bupd, lidx))

    sums, cnts = _pool(batch, h2, n, bn=400)
    w_out_p = jnp.pad(W_out, ((0, 0), (0, LN - c_out)))
    b_out_p = jnp.tile(b_out.reshape(1, -1), (8, 1))
    b_out_p = jnp.pad(b_out_p, ((0, 0), (0, LN - c_out)))
    out = _head(sums, cnts, w_out_p, b_out_p)
    return out[:, :c_out]


# speedup vs baseline: 2.6198x; 1.0185x over previous
import functools

import jax
import jax.numpy as jnp
from jax import lax
from jax.experimental import pallas as pl
from jax.experimental.pallas import tpu as pltpu
from jax.experimental.pallas import tpu_sc as plsc

G = 64
NC = 2
NS = 16
CH = 48
SK = 15
LN = 128


def _mm_stacked_body(x_ref, w_ref, b_ref, o_ref):
    o_ref[...] = (
        jnp.dot(x_ref[...], w_ref[0], preferred_element_type=jnp.float32)
        + b_ref[0]
    )


def _mm_stackedT_body(xt_ref, w_ref, b_ref, o_ref):
    o_ref[...] = (
        lax.dot_general(xt_ref[...], w_ref[0], (((0,), (0,)), ((), ())),
                        preferred_element_type=jnp.float32)
        + b_ref[0]
    )


def _stacked_matmulT(xt, wcat, bcat, bm, ln=LN):
    k, m = xt.shape
    nj = wcat.shape[1] // ln
    ni = m // bm
    w3 = wcat.reshape(k, nj, ln).transpose(1, 0, 2)
    return pl.pallas_call(
        _mm_stackedT_body,
        grid=(ni, nj),
        in_specs=[
            pl.BlockSpec((k, bm), lambda i, j: (0, i)),
            pl.BlockSpec((1, k, ln), lambda i, j: (j, 0, 0)),
            pl.BlockSpec((1, 1, ln), lambda i, j: (j, 0, 0)),
        ],
        out_specs=pl.BlockSpec((bm, ln), lambda i, j, ni=ni: (j * ni + i, 0)),
        out_shape=jax.ShapeDtypeStruct((nj * m, ln), jnp.float32),
    )(xt, w3, bcat.reshape(nj, 1, ln))


def _mm_stacked2_body(x_ref, y_ref, w_ref, u_ref, o_ref):
    o_ref[...] = (
        jnp.dot(x_ref[...], w_ref[0], preferred_element_type=jnp.float32)
        + jnp.dot(y_ref[...], u_ref[0], preferred_element_type=jnp.float32)
    )


def _stacked_matmul2(xp, yp, wcat, ucat, bm, ln=LN):
    m, k1 = xp.shape
    k2 = yp.shape[1]
    nj = wcat.shape[1] // ln
    ni = m // bm
    w3 = wcat.reshape(k1, nj, ln).transpose(1, 0, 2)
    u3 = ucat.reshape(k2, nj, ln).transpose(1, 0, 2)
    return pl.pallas_call(
        _mm_stacked2_body,
        grid=(ni, nj),
        in_specs=[
            pl.BlockSpec((bm, k1), lambda i, j: (i, 0)),
            pl.BlockSpec((bm, k2), lambda i, j: (i, 0)),
            pl.BlockSpec((1, k1, ln), lambda i, j: (j, 0, 0)),
            pl.BlockSpec((1, k2, ln), lambda i, j: (j, 0, 0)),
        ],
        out_specs=pl.BlockSpec((bm, ln), lambda i, j, ni=ni: (j * ni + i, 0)),
        out_shape=jax.ShapeDtypeStruct((nj * m, ln), jnp.float32),
    )(xp, yp, w3, u3)


def _stacked_matmul(xp, wcat, bcat, bm, ln=LN):
    m, k = xp.shape
    nj = wcat.shape[1] // ln
    ni = m // bm
    w3 = wcat.reshape(k, nj, ln).transpose(1, 0, 2)
    return pl.pallas_call(
        _mm_stacked_body,
        grid=(nj, ni),
        in_specs=[
            pl.BlockSpec((bm, k), lambda j, i: (i, 0)),
            pl.BlockSpec((1, k, ln), lambda j, i: (j, 0, 0)),
            pl.BlockSpec((1, 1, ln), lambda j, i: (j, 0, 0)),
        ],
        out_specs=pl.BlockSpec((bm, ln), lambda j, i, ni=ni: (j * ni + i, 0)),
        out_shape=jax.ShapeDtypeStruct((nj * m, ln), jnp.float32),
    )(xp, w3, bcat.reshape(nj, 1, ln))


def _upd_body(h_ref, alo_ref, ahi_ref, wh_ref, wlo_ref, whi_ref, b_ref,
              o_ref):
    acc = jnp.dot(h_ref[...], wh_ref[...], preferred_element_type=jnp.float32)
    acc += jnp.dot(alo_ref[...], wlo_ref[...],
                   preferred_element_type=jnp.float32)
    acc += jnp.dot(ahi_ref[...], whi_ref[...],
                   preferred_element_type=jnp.float32)
    o_ref[...] = jnp.maximum(acc + b_ref[0], 0.0)


def _update_matmul(h, agg, w_upd, b_upd, bm):
    npad, fh = h.shape
    hout = w_upd.shape[1]
    ni = npad // bm
    nc = hout // LN
    wh = w_upd[:fh]
    wlo = w_upd[fh:fh + LN]
    whi = w_upd[fh + LN:]
    return pl.pallas_call(
        _upd_body,
        grid=(ni, nc),
        in_specs=[
            pl.BlockSpec((bm, fh), lambda i, c: (i, 0)),
            pl.BlockSpec((bm, LN), lambda i, c: (i, 0)),
            pl.BlockSpec((bm, LN), lambda i, c, ni=ni: (ni + i, 0)),
            pl.BlockSpec((fh, LN), lambda i, c: (0, c)),
            pl.BlockSpec((LN, LN), lambda i, c: (0, c)),
            pl.BlockSpec((LN, LN), lambda i, c: (0, c)),
            pl.BlockSpec((1, 1, LN), lambda i, c: (c, 0, 0)),
        ],
        out_specs=pl.BlockSpec((bm, LN), lambda i, c: (i, c)),
        out_shape=jax.ShapeDtypeStruct((npad, hout), jnp.float32),
    )(h, agg, agg, wh, wlo, whi, b_upd.reshape(nc, 1, LN))


def _pool_body(b3_ref, h_ref, sum_ref, cnt_ref):
    i = pl.program_id(0)

    @pl.when(i == 0)
    def _():
        sum_ref[...] = jnp.zeros_like(sum_ref)
        cnt_ref[...] = jnp.zeros_like(cnt_ref)

    bids = b3_ref[0]
    ids = lax.broadcasted_iota(jnp.int32, (G, bids.shape[1]), 0)
    oh = (bids == ids).astype(jnp.float32)
    sum_ref[...] += jnp.dot(oh, h_ref[...], preferred_element_type=jnp.float32)
    cnt_ref[...] += jnp.sum(oh, axis=1, keepdims=True)


def _pool(batch, h, n, bn):
    nb = n // bn
    fh = h.shape[1]
    batch3 = batch.reshape(nb, 1, bn)
    return pl.pallas_call(
        _pool_body,
        grid=(nb,),
        in_specs=[
            pl.BlockSpec((1, 1, bn), lambda i: (i, 0, 0)),
            pl.BlockSpec((bn, fh), lambda i: (i, 0)),
        ],
        out_specs=[
            pl.BlockSpec((G, fh), lambda i: (0, 0)),
            pl.BlockSpec((G, LN), lambda i: (0, 0)),
        ],
        out_shape=[
            jax.ShapeDtypeStruct((G, fh), jnp.float32),
            jax.ShapeDtypeStruct((G, LN), jnp.float32),
        ],
    )(batch3, h)


def _head_body(s_ref, c_ref, w_ref, b_ref, o_ref):
    cnt = c_ref[:, :1]
    pooled = s_ref[...] / jnp.clip(cnt, 1.0, None)
    o_ref[...] = (
        jnp.dot(pooled, w_ref[...], preferred_element_type=jnp.float32)
        + b_ref[0:1, :]
    )


def _head(sums, cnts, w_out_p, b_out_p):
    fh = sums.shape[1]
    return pl.pallas_call(
        _head_body,
        out_shape=jax.ShapeDtypeStruct((G, LN), jnp.float32),
    )(sums, cnts, w_out_p, b_out_p)


def _make_edge_kernel(npad, etot, nch):
    nsc = nch // SK
    eps = nch * CH
    rows_per = npad // NS
    ZB = 128
    nz = rows_per // ZB

    def body(t_hbm, eaw_hbm, src_hbm, dst_hbm, zero_hbm, lidx_hbm, out_hbm,
             idx_ag, idx_dr, idx_bg,
             a0_v, a1_v, a2_v, b0_v, b1_v, e0_v, e1_v,
             ga0, ga1, ga2, gb0, gb1, ge0, ge1, sc0, sc1, sc2,
             shared):
        a_bufs = (a0_v, a1_v, a2_v)
        b_bufs = (b0_v, b1_v)
        e_bufs = (e0_v, e1_v)
        ga_sem = (ga0, ga1, ga2)
        gb_sem = (gb0, gb1)
        ge_sem = (ge0, ge1)
        sc_sem = (sc0, sc1, sc2)

        c = lax.axis_index("c")
        s = lax.axis_index("s")
        for k in range(nz):
            pltpu.sync_copy(
                zero_hbm, shared.at[pl.ds(s * rows_per + k * ZB, ZB)])
        plsc.subcore_barrier()

        pltpu.sync_copy(lidx_hbm, idx_ag.at[SK, pl.ds(0, 16)])
        layer = idx_ag[SK, pl.ds(0, 16)][0]
        a_off = c * npad
        b_off = (2 + c) * npad
        ebase = (2 * layer + c) * etot + s * eps

        def super_body(g, carry):
            pltpu.sync_copy(src_hbm.at[s, g], idx_ag.at[pl.ds(0, SK)])
            pltpu.sync_copy(dst_hbm.at[s, g], idx_dr)

            def off_body(r, cc):
                for v in range(CH // 16):
                    sl = pl.ds(v * 16, 16)
                    idx_bg[r, sl] = idx_dr[r, sl] + b_off
                    idx_ag[r, sl] = idx_ag[r, sl] + a_off
                return cc

            lax.fori_loop(0, SK, off_body, 0)

            gd = {}
            sd = {}
            eg = ebase + g * (SK * CH)

            def gfire(k):
                if k >= 3:
                    sd[k - 3].wait()
                gd[k] = (
                    pltpu.async_copy(
                        t_hbm.at[idx_ag.at[k]], a_bufs[k % 3],
                        ga_sem[k % 3]),
                    pltpu.async_copy(
                        t_hbm.at[idx_bg.at[k]], b_bufs[k % 2],
                        gb_sem[k % 2]),
                    pltpu.async_copy(
                        eaw_hbm.at[pl.ds(eg + k * CH, CH)], e_bufs[k % 2],
                        ge_sem[k % 2]),
                )

            gfire(0)
            gfire(1)
            for k in range(SK):
                a_v, b_v, e_v = a_bufs[k % 3], b_bufs[k % 2], e_bufs[k % 2]
                for d in gd.pop(k):
                    d.wait()

                def comp(ei, cc, a_v=a_v, b_v=b_v, e_v=e_v):
                    for v in range(LN // 16):
                        sl = pl.ds(v * 16, 16)
                        a_v[ei, sl] = jnp.maximum(
                            a_v[ei, sl] + b_v[ei, sl] + e_v[ei, sl], 0.0)
                    return cc

                lax.fori_loop(0, CH, comp, 0)
                sd[k] = pltpu.async_copy(
                    a_v, shared.at[idx_dr.at[k]], sc_sem[k % 3], add=True)
                if k + 2 < SK:
                    gfire(k + 2)
            for k in range(SK - 3, SK):
                sd[k].wait()
            return carry

        lax.fori_loop(0, nsc, super_body, 0)
        plsc.subcore_barrier()
        for k in range(nz):
            off = s * rows_per + k * ZB
            pltpu.sync_copy(
                shared.at[pl.ds(off, ZB)],
                out_hbm.at[pl.ds(c * npad + off, ZB)])

    mesh = plsc.VectorSubcoreMesh(
        core_axis_name="c", subcore_axis_name="s",
        num_cores=NC, num_subcores=NS)
    return pl.kernel(
        body,
        out_type=jax.ShapeDtypeStruct((2 * npad, LN), jnp.float32),
        mesh=mesh,
        scratch_types=[
            pltpu.VMEM((SK + 1, CH), jnp.int32),
            pltpu.VMEM((SK, CH), jnp.int32),
            pltpu.VMEM((SK, CH), jnp.int32),
            pltpu.VMEM((CH, LN), jnp.float32),
            pltpu.VMEM((CH, LN), jnp.float32),
            pltpu.VMEM((CH, LN), jnp.float32),
            pltpu.VMEM((CH, LN), jnp.float32),
            pltpu.VMEM((CH, LN), jnp.float32),
            pltpu.VMEM((CH, LN), jnp.float32),
            pltpu.VMEM((CH, LN), jnp.float32),
            pltpu.SemaphoreType.DMA,
            pltpu.SemaphoreType.DMA,
            pltpu.SemaphoreType.DMA,
            pltpu.SemaphoreType.DMA,
            pltpu.SemaphoreType.DMA,
            pltpu.SemaphoreType.DMA,
            pltpu.SemaphoreType.DMA,
            pltpu.SemaphoreType.DMA,
            pltpu.SemaphoreType.DMA,
            pltpu.SemaphoreType.DMA,
            pltpu.VMEM_SHARED((npad, LN), jnp.float32),
        ],
    )


def _round_up(a, b):
    return -(-a // b) * b


def kernel(x, edge_index, batch, edge_attr, pos, W_msg1, b_msg1, W_upd1,
           b_upd1, W_msg2, b_msg2, W_upd2, b_upd2, W_out, b_out):
    f32 = jnp.float32
    n, f = x.shape
    e = edge_index.shape[1]
    ed = edge_attr.shape[1]
    h = W_upd1.shape[1]
    c_out = W_out.shape[1]

    npad = _round_up(n + 1, NS * 128)
    etot = _round_up(e, NS * CH * SK)
    nch = etot // (NS * CH)
    kp = _round_up(f + 3, 128)

    src = edge_index[0]
    dst = edge_index[1]
    srcp = jnp.concatenate([src, jnp.zeros((etot - e,), jnp.int32)])
    dstp = jnp.concatenate([dst, jnp.full((etot - e,), n, jnp.int32)])
    src3 = srcp.reshape(NS, nch // SK, SK, CH)
    dst3 = dstp.reshape(NS, nch // SK, SK, CH)

    posp = jnp.pad(pos, ((0, npad - n), (0, 0)))
    xpad = jnp.pad(x, ((0, npad - n), (0, 0)))
    zero128 = jnp.zeros((128, LN), f32)

    posp8 = jnp.pad(posp, ((0, 0), (0, 5)))

    def node_tables(hh, w_msg, fh):
        hw = w_msg.shape[1]
        wx = w_msg[:fh]
        wp = w_msg[fh + ed:]
        wcat = jnp.concatenate([wx, jnp.zeros((fh, hw), f32)], axis=1)
        wp8 = jnp.pad(jnp.concatenate([-wp, wp], axis=1), ((0, 5), (0, 0)))
        return _stacked_matmul2(hh, posp8, wcat, wp8, bm=512)

    eat = jnp.pad(edge_attr.T, ((0, 0), (0, etot - e)))
    wecat = jnp.concatenate([W_msg1[f:f + ed], W_msg2[h:h + ed]], axis=1)
    becat = jnp.concatenate([b_msg1, b_msg2])
    eaw_both = _stacked_matmulT(eat, wecat, becat, bm=2560)

    edge_k = _make_edge_kernel(npad, etot, nch)

    wmsg = jnp.stack([W_msg1, W_msg2])
    wupd = jnp.stack([W_upd1, W_upd2])
    bupd = jnp.stack([b_upd1, b_upd2])
    lidx = jnp.stack([jnp.zeros((16,), jnp.int32),
                      jnp.ones((16,), jnp.int32)])

    def layer_step(hcur, ws):
        wm, wu, bu, li = ws
        t = node_tables(hcur, wm, f)
        agg = edge_k(t, eaw_both, src3, dst3, zero128, li)
        hnext = _update_matmul(hcur, agg, wu, bu, bm=512)
        return hnext, 0.0

    h2, _ = lax.scan(layer_step, xpad, (wmsg, wupd, bupd, lidx))

    sums, cnts = _pool(batch, h2, n, bn=400)
    w_out_p = jnp.pad(W_out, ((0, 0), (0, LN - c_out)))
    b_out_p = jnp.tile(b_out.reshape(1, -1), (8, 1))
    b_out_p = jnp.pad(b_out_p, ((0, 0), (0, LN - c_out)))
    out = _head(sums, cnts, w_out_p, b_out_p)
    return out[:, :c_out]
